# Initial kernel scaffold; baseline (speedup 1.0000x reference)
import jax
import jax.numpy as jnp
from jax.experimental import pallas as pl


def kernel(attention_mask, past_key_values_length, weight):
    raise NotImplementedError("write your pallas kernel here")


# speedup vs baseline: 2.4305x; 2.4305x over previous
import functools

import jax
import jax.numpy as jnp
from jax import lax
from jax.experimental import pallas as pl
from jax.experimental.pallas import tpu as pltpu
from jax.experimental.pallas import tpu_sc as plsc

NUM_EMBEDDINGS = 8192
EMBEDDING_DIM = 768
POS_OFFSET = 2
BATCH = 4
SEQ_LEN = 8192

_V = NUM_EMBEDDINGS + POS_OFFSET
_NW = 32
_SPAN = (BATCH * SEQ_LEN) // _NW
_G = 64
_NCH = _SPAN // _G


def _pid_body(mask_ref, idx_ref):
    m = mask_ref[...]
    s = m
    sh = 1
    while sh < SEQ_LEN:
        zeros = jnp.zeros((BATCH, sh), jnp.int32)
        s = s + jnp.concatenate([zeros, s[:, : SEQ_LEN - sh]], axis=1)
        sh *= 2
    ids = s * m + 1
    idx_ref[...] = jnp.minimum(jnp.maximum(ids, 0), _V - 1)


_pid = pl.pallas_call(
    _pid_body,
    out_shape=jax.ShapeDtypeStruct((BATCH, SEQ_LEN), jnp.int32),
)


def _sc_body(idx_hbm, weight_hbm, out_hbm, idx_v, rows_a, rows_b, sem_a, sem_b):
    cid = lax.axis_index("c")
    sid = lax.axis_index("s")
    wid = cid * 16 + sid
    out_base = wid * _SPAN

    pltpu.sync_copy(idx_hbm.at[wid], idx_v)

    pltpu.async_copy(weight_hbm.at[idx_v.at[0]], rows_a, sem_a)

    def _gather_step(j, carry):
        j = j * 2
        pltpu.async_copy(weight_hbm.at[idx_v.at[j + 1]], rows_b, sem_b)
        pltpu.make_async_copy(weight_hbm.at[idx_v.at[j]], rows_a, sem_a).wait()
        pltpu.sync_copy(rows_a, out_hbm.at[pl.ds(out_base + j * _G, _G)])
        nxt_j = jnp.minimum(j + 2, _NCH - 2)
        pltpu.async_copy(weight_hbm.at[idx_v.at[nxt_j]], rows_a, sem_a)
        pltpu.make_async_copy(weight_hbm.at[idx_v.at[j + 1]], rows_b, sem_b).wait()
        pltpu.sync_copy(rows_b, out_hbm.at[pl.ds(out_base + (j + 1) * _G, _G)])
        return carry

    lax.fori_loop(0, _NCH // 2, _gather_step, 0, unroll=False)
    pltpu.make_async_copy(weight_hbm.at[idx_v.at[_NCH - 2]], rows_a, sem_a).wait()


@functools.partial(
    pl.kernel,
    mesh=plsc.VectorSubcoreMesh(core_axis_name="c", subcore_axis_name="s"),
    out_type=jax.ShapeDtypeStruct((BATCH * SEQ_LEN, EMBEDDING_DIM), jnp.float32),
    scratch_types=[
        pltpu.VMEM((_NCH, _G), jnp.int32),
        pltpu.VMEM((_G, EMBEDDING_DIM), jnp.float32),
        pltpu.VMEM((_G, EMBEDDING_DIM), jnp.float32),
        pltpu.SemaphoreType.DMA,
        pltpu.SemaphoreType.DMA,
    ],
)
def _embed_gather(idx_hbm, weight_hbm, out_hbm, idx_v, rows_a, rows_b,
                  sem_a, sem_b):
    _sc_body(idx_hbm, weight_hbm, out_hbm, idx_v, rows_a, rows_b, sem_a, sem_b)


def kernel(attention_mask, past_key_values_length, weight):
    del past_key_values_length
    idx = _pid(attention_mask.astype(jnp.int32))
    out = _embed_gather(idx.reshape(_NW, _NCH, _G), weight)
    return out.reshape(BATCH, SEQ_LEN, EMBEDDING_DIM)


# speedup vs baseline: 2.4575x; 1.0111x over previous
import functools

import jax
import jax.numpy as jnp
from jax import lax
from jax.experimental import pallas as pl
from jax.experimental.pallas import tpu as pltpu
from jax.experimental.pallas import tpu_sc as plsc

NUM_EMBEDDINGS = 8192
EMBEDDING_DIM = 768
POS_OFFSET = 2
BATCH = 4
SEQ_LEN = 8192

_V = NUM_EMBEDDINGS + POS_OFFSET
_NW = 32
_SPAN = (BATCH * SEQ_LEN) // _NW
_G = 32
_NCH = _SPAN // _G


def _pid_body(mask_ref, idx_ref):
    m = mask_ref[...]
    s = m
    sh = 1
    while sh < SEQ_LEN:
        zeros = jnp.zeros((BATCH, sh), jnp.int32)
        s = s + jnp.concatenate([zeros, s[:, : SEQ_LEN - sh]], axis=1)
        sh *= 2
    ids = s * m + 1
    idx_ref[...] = jnp.minimum(jnp.maximum(ids, 0), _V - 1)


_pid = pl.pallas_call(
    _pid_body,
    out_shape=jax.ShapeDtypeStruct((BATCH, SEQ_LEN), jnp.int32),
)


_NBUF = 4
_NGRP = _NCH // _NBUF


def _sc_body(idx_hbm, weight_hbm, out_hbm, idx_v, rows, sem_r, sem_w):
    cid = lax.axis_index("c")
    sid = lax.axis_index("s")
    wid = cid * 16 + sid
    out_base = wid * _SPAN

    pltpu.sync_copy(idx_hbm.at[wid], idx_v)

    def _wait_read(b):
        pltpu.make_async_copy(weight_hbm.at[idx_v.at[0]], rows[b], sem_r[b]).wait()

    def _wait_write(b):
        pltpu.make_async_copy(rows[b], out_hbm.at[pl.ds(out_base, _G)],
                              sem_w[b]).wait()

    pltpu.async_copy(weight_hbm.at[idx_v.at[0]], rows[0], sem_r[0])
    pltpu.async_copy(weight_hbm.at[idx_v.at[1]], rows[1], sem_r[1])

    def _group(g, carry):
        for b in range(_NBUF):
            j = g * _NBUF + b
            _wait_read(b)
            pltpu.async_copy(rows[b], out_hbm.at[pl.ds(out_base + j * _G, _G)],
                             sem_w[b])
            b2 = (b + 2) % _NBUF
            if b < 2:
                @pl.when(g > 0)
                def _():
                    _wait_write(b2)

                pltpu.async_copy(weight_hbm.at[idx_v.at[j + 2]], rows[b2],
                                 sem_r[b2])
            else:
                @pl.when(g < _NGRP - 1)
                def _():
                    _wait_write(b2)
                    pltpu.async_copy(weight_hbm.at[idx_v.at[j + 2]], rows[b2],
                                     sem_r[b2])
        return carry

    lax.fori_loop(0, _NGRP, _group, 0, unroll=False)
    for b in range(_NBUF):
        _wait_write(b)


@functools.partial(
    pl.kernel,
    mesh=plsc.VectorSubcoreMesh(core_axis_name="c", subcore_axis_name="s"),
    out_type=jax.ShapeDtypeStruct((BATCH * SEQ_LEN, EMBEDDING_DIM), jnp.float32),
    scratch_types=[
        pltpu.VMEM((_NCH, _G), jnp.int32),
        [pltpu.VMEM((_G, EMBEDDING_DIM), jnp.float32)] * _NBUF,
        [pltpu.SemaphoreType.DMA] * _NBUF,
        [pltpu.SemaphoreType.DMA] * _NBUF,
    ],
)
def _embed_gather(idx_hbm, weight_hbm, out_hbm, idx_v, rows, sem_r, sem_w):
    _sc_body(idx_hbm, weight_hbm, out_hbm, idx_v, rows, sem_r, sem_w)


def kernel(attention_mask, past_key_values_length, weight):
    del past_key_values_length
    idx = _pid(attention_mask.astype(jnp.int32))
    out = _embed_gather(idx.reshape(_NW, _NCH, _G), weight)
    return out.reshape(BATCH, SEQ_LEN, EMBEDDING_DIM)


# speedup vs baseline: 3.4218x; 1.3924x over previous
import functools

import jax
import jax.numpy as jnp
from jax import lax
from jax.experimental import pallas as pl
from jax.experimental.pallas import tpu as pltpu
from jax.experimental.pallas import tpu_sc as plsc

NUM_EMBEDDINGS = 8192
EMBEDDING_DIM = 768
POS_OFFSET = 2
BATCH = 4
SEQ_LEN = 8192

_V = NUM_EMBEDDINGS + POS_OFFSET
_NW = 32
_SPAN = SEQ_LEN // _NW
_G = 32
_NCH = _SPAN // _G
_NBUF = 4


def _pid_body(mask_ref, idx_ref, flag_ref):
    m = mask_ref[...]
    s = m
    sh = 1
    while sh < SEQ_LEN:
        zeros = jnp.zeros((BATCH, sh), jnp.int32)
        s = s + jnp.concatenate([zeros, s[:, : SEQ_LEN - sh]], axis=1)
        sh *= 2
    ids = s * m + 1
    ids = jnp.minimum(jnp.maximum(ids, 0), _V - 1)
    idx_ref[...] = ids
    eq = (ids == ids[0:1, :]).astype(jnp.int32)
    flag_ref[...] = jnp.full((8, 128), jnp.min(eq), jnp.int32)


_pid = pl.pallas_call(
    _pid_body,
    out_shape=(
        jax.ShapeDtypeStruct((BATCH, SEQ_LEN), jnp.int32),
        jax.ShapeDtypeStruct((8, 128), jnp.int32),
    ),
)


def _sc_body(flag_hbm, idx_hbm, weight_hbm, out_hbm, flag_v, idx_v, rows,
             sem_r, sem_w):
    cid = lax.axis_index("c")
    sid = lax.axis_index("s")
    wid = cid * 16 + sid

    pltpu.sync_copy(flag_hbm.at[0], flag_v)
    for b in range(BATCH):
        pltpu.sync_copy(idx_hbm.at[b, wid], idx_v.at[b])
    shared = flag_v[pl.ds(0, 16)][0] != 0

    def _out_slice(b, j):
        return out_hbm.at[pl.ds(b * SEQ_LEN + wid * _SPAN + j * _G, _G)]

    def _fire_read(j):
        k = j % _NBUF
        pltpu.async_copy(weight_hbm.at[idx_v.at[0, j]], rows[k], sem_r[k])

    def _wait_read(j):
        k = j % _NBUF
        pltpu.make_async_copy(weight_hbm.at[idx_v.at[0, j]], rows[k],
                              sem_r[k]).wait()

    def _fire_writes(j):
        k = j % _NBUF
        for b in range(BATCH):
            pltpu.async_copy(rows[k], _out_slice(b, j), sem_w[k])

    def _wait_writes(j):
        k = j % _NBUF
        for b in range(BATCH):
            pltpu.make_async_copy(rows[k], _out_slice(b, j), sem_w[k]).wait()

    @pl.when(shared)
    def _fan_out():
        _fire_read(0)
        _fire_read(1)
        for j in range(_NCH):
            _wait_read(j)
            _fire_writes(j)
            if j + 2 < _NCH:
                if j - 2 >= 0:
                    _wait_writes(j - 2)
                _fire_read(j + 2)
        for j in range(_NCH - 4, _NCH):
            _wait_writes(j)

    @pl.when(jnp.logical_not(shared))
    def _full_gather():
        for b in range(BATCH):
            pltpu.async_copy(weight_hbm.at[idx_v.at[b, 0]], rows[0], sem_r[0])

            def _step(g, carry, b=b):
                j0 = g * 2
                pltpu.async_copy(weight_hbm.at[idx_v.at[b, j0 + 1]], rows[1],
                                 sem_r[1])
                pltpu.make_async_copy(weight_hbm.at[idx_v.at[b, 0]], rows[0],
                                      sem_r[0]).wait()
                pltpu.sync_copy(rows[0], out_hbm.at[
                    pl.ds(b * SEQ_LEN + wid * _SPAN + j0 * _G, _G)])
                jn = jnp.minimum(j0 + 2, _NCH - 1)
                pltpu.async_copy(weight_hbm.at[idx_v.at[b, jn]], rows[0],
                                 sem_r[0])
                pltpu.make_async_copy(weight_hbm.at[idx_v.at[b, 0]], rows[1],
                                      sem_r[1]).wait()
                pltpu.sync_copy(rows[1], out_hbm.at[
                    pl.ds(b * SEQ_LEN + wid * _SPAN + (j0 + 1) * _G, _G)])
                return carry

            lax.fori_loop(0, _NCH // 2, _step, 0, unroll=False)
            pltpu.make_async_copy(weight_hbm.at[idx_v.at[b, 0]], rows[0],
                                  sem_r[0]).wait()


@functools.partial(
    pl.kernel,
    mesh=plsc.VectorSubcoreMesh(core_axis_name="c", subcore_axis_name="s"),
    out_type=jax.ShapeDtypeStruct((BATCH * SEQ_LEN, EMBEDDING_DIM), jnp.float32),
    scratch_types=[
        pltpu.VMEM((128,), jnp.int32),
        pltpu.VMEM((BATCH, _NCH, _G), jnp.int32),
        [pltpu.VMEM((_G, EMBEDDING_DIM), jnp.float32)] * _NBUF,
        [pltpu.SemaphoreType.DMA] * _NBUF,
        [pltpu.SemaphoreType.DMA] * _NBUF,
    ],
)
def _embed_gather(flag_hbm, idx_hbm, weight_hbm, out_hbm, flag_v, idx_v, rows,
                  sem_r, sem_w):
    _sc_body(flag_hbm, idx_hbm, weight_hbm, out_hbm, flag_v, idx_v, rows,
             sem_r, sem_w)


def kernel(attention_mask, past_key_values_length, weight):
    del past_key_values_length
    idx, flag = _pid(attention_mask.astype(jnp.int32))
    out = _embed_gather(flag, idx.reshape(BATCH, _NW, _NCH, _G), weight)
    return out.reshape(BATCH, SEQ_LEN, EMBEDDING_DIM)
